# baseline (speedup 1.0000x reference)
import functools

import jax
import jax.numpy as jnp
from jax.experimental import pallas as pl

NA = 3
NO = 85


def _head_kernel(x_ref, w_ref, b_ref, o_ref):
    res = jax.lax.dot_general(
        x_ref[0], w_ref[0],
        dimension_numbers=(((0,), (0,)), ((), ())),
        preferred_element_type=jnp.float32,
    )
    o_ref[0, 0] = res + b_ref[0]


@functools.partial(jax.jit, static_argnames=("hw_chunk",))
def _head(x, W, b, hw_chunk):
    bs, C, ny, nx = x.shape
    hw = ny * nx
    nchunks = hw // hw_chunk
    xr = x.reshape(bs, C, hw)
    wt = W.reshape(NA, NO, C).transpose(0, 2, 1)
    br = b.reshape(NA, 1, NO)
    out = pl.pallas_call(
        _head_kernel,
        grid=(bs, nchunks, NA),
        in_specs=[
            pl.BlockSpec((1, C, hw_chunk), lambda bidx, c, a: (bidx, 0, c)),
            pl.BlockSpec((1, C, NO), lambda bidx, c, a: (a, 0, 0)),
            pl.BlockSpec((1, 1, NO), lambda bidx, c, a: (a, 0, 0)),
        ],
        out_specs=pl.BlockSpec((1, 1, hw_chunk, NO),
                               lambda bidx, c, a: (bidx, a, c, 0)),
        out_shape=jax.ShapeDtypeStruct((bs, NA, hw, NO), jnp.float32),
    )(xr, wt, br)
    return out.reshape(bs, NA, ny, nx, NO)


def kernel(x0, x1, x2, W0, b0, W1, b1, W2, b2):
    return (_head(x0, W0, b0, hw_chunk=1024),
            _head(x1, W1, b1, hw_chunk=512),
            _head(x2, W2, b2, hw_chunk=256))


# speedup vs baseline: 1.9947x; 1.9947x over previous
import functools

import jax
import jax.numpy as jnp
from jax.experimental import pallas as pl

NA = 3
NO = 85


def _head_kernel(nb, x_ref, w_ref, b_ref, o_ref):
    for i in range(nb):
        res = jax.lax.dot_general(
            x_ref[i], w_ref[...],
            dimension_numbers=(((0,), (0,)), ((), ())),
            preferred_element_type=jnp.float32,
        )
        res = res + b_ref[0]
        for a in range(NA):
            o_ref[i, a] = res[:, a * NO:(a + 1) * NO]


@functools.partial(jax.jit, static_argnames=("nb",))
def _head(x, W, b, nb):
    bs, C, ny, nx = x.shape
    hw = ny * nx
    xr = x.reshape(bs, C, hw)
    wt = W.T
    br = b.reshape(1, NA * NO)
    out = pl.pallas_call(
        functools.partial(_head_kernel, nb),
        grid=(bs // nb,),
        in_specs=[
            pl.BlockSpec((nb, C, hw), lambda g: (g, 0, 0)),
            pl.BlockSpec((C, NA * NO), lambda g: (0, 0)),
            pl.BlockSpec((1, NA * NO), lambda g: (0, 0)),
        ],
        out_specs=pl.BlockSpec((nb, NA, hw, NO), lambda g: (g, 0, 0, 0)),
        out_shape=jax.ShapeDtypeStruct((bs, NA, hw, NO), jnp.float32),
    )(xr, wt, br)
    return out.reshape(bs, NA, ny, nx, NO)


def kernel(x0, x1, x2, W0, b0, W1, b1, W2, b2):
    return (_head(x0, W0, b0, nb=1),
            _head(x1, W1, b1, nb=2),
            _head(x2, W2, b2, nb=4))


# speedup vs baseline: 2.0253x; 1.0154x over previous
import functools

import jax
import jax.numpy as jnp
from jax.experimental import pallas as pl

NA = 3
NO = 85


def _head_kernel(nb, x_ref, w_ref, b_ref, o_ref):
    for i in range(nb):
        res = jax.lax.dot_general(
            x_ref[i], w_ref[...],
            dimension_numbers=(((0,), (0,)), ((), ())),
            preferred_element_type=jnp.float32,
        )
        res = res + b_ref[0]
        for a in range(NA):
            o_ref[i, a] = res[:, a * NO:(a + 1) * NO]


@functools.partial(jax.jit, static_argnames=("nb",))
def _head(x, W, b, nb):
    bs, C, ny, nx = x.shape
    hw = ny * nx
    xr = x.reshape(bs, C, hw)
    wt = W.T
    br = b.reshape(1, NA * NO)
    out = pl.pallas_call(
        functools.partial(_head_kernel, nb),
        grid=(bs // nb,),
        in_specs=[
            pl.BlockSpec((nb, C, hw), lambda g: (g, 0, 0)),
            pl.BlockSpec((C, NA * NO), lambda g: (0, 0)),
            pl.BlockSpec((1, NA * NO), lambda g: (0, 0)),
        ],
        out_specs=pl.BlockSpec((nb, NA, hw, NO), lambda g: (g, 0, 0, 0)),
        out_shape=jax.ShapeDtypeStruct((bs, NA, hw, NO), jnp.float32),
    )(xr, wt, br)
    return out.reshape(bs, NA, ny, nx, NO)


def kernel(x0, x1, x2, W0, b0, W1, b1, W2, b2):
    return (_head(x0, W0, b0, nb=2),
            _head(x1, W1, b1, nb=4),
            _head(x2, W2, b2, nb=8))


# speedup vs baseline: 2.0285x; 1.0016x over previous
import functools

import jax
import jax.numpy as jnp
from jax.experimental import pallas as pl
from jax.experimental.pallas import tpu as pltpu

NA = 3
NO = 85


def _head_kernel(nb, x_ref, w_ref, b_ref, o_ref):
    for i in range(nb):
        res = jax.lax.dot_general(
            x_ref[i], w_ref[...],
            dimension_numbers=(((0,), (0,)), ((), ())),
            preferred_element_type=jnp.float32,
        )
        res = res + b_ref[0]
        for a in range(NA):
            o_ref[i, a] = res[:, a * NO:(a + 1) * NO]


@functools.partial(jax.jit, static_argnames=("nb",))
def _head(x, W, b, nb):
    bs, C, ny, nx = x.shape
    hw = ny * nx
    xr = x.reshape(bs, C, hw)
    wt = W.T
    br = b.reshape(1, NA * NO)
    out = pl.pallas_call(
        functools.partial(_head_kernel, nb),
        grid=(bs // nb,),
        in_specs=[
            pl.BlockSpec((nb, C, hw), lambda g: (g, 0, 0)),
            pl.BlockSpec((C, NA * NO), lambda g: (0, 0)),
            pl.BlockSpec((1, NA * NO), lambda g: (0, 0)),
        ],
        out_specs=pl.BlockSpec((nb, NA, hw, NO), lambda g: (g, 0, 0, 0)),
        out_shape=jax.ShapeDtypeStruct((bs, NA, hw, NO), jnp.float32),
        compiler_params=pltpu.CompilerParams(
            dimension_semantics=("parallel",)),
    )(xr, wt, br)
    return out.reshape(bs, NA, ny, nx, NO)


def kernel(x0, x1, x2, W0, b0, W1, b1, W2, b2):
    return (_head(x0, W0, b0, nb=2),
            _head(x1, W1, b1, nb=4),
            _head(x2, W2, b2, nb=8))


# speedup vs baseline: 2.0341x; 1.0027x over previous
import jax
import jax.numpy as jnp
from jax.experimental import pallas as pl
from jax.experimental.pallas import tpu as pltpu

NA = 3
NO = 85


def _detect_kernel(x0_ref, x1_ref, x2_ref, w0_ref, w1_ref, w2_ref,
                   b_ref, o0_ref, o1_ref, o2_ref):
    for lvl, (x_ref, w_ref, o_ref) in enumerate(
            ((x0_ref, w0_ref, o0_ref),
             (x1_ref, w1_ref, o1_ref),
             (x2_ref, w2_ref, o2_ref))):
        res = jax.lax.dot_general(
            x_ref[0], w_ref[...],
            dimension_numbers=(((0,), (0,)), ((), ())),
            preferred_element_type=jnp.float32,
        )
        res = res + b_ref[lvl]
        for a in range(NA):
            o_ref[0, a] = res[:, a * NO:(a + 1) * NO]


def kernel(x0, x1, x2, W0, b0, W1, b1, W2, b2):
    bs = x0.shape[0]
    shapes = [x.shape for x in (x0, x1, x2)]
    hws = [ny * nx for (_, _, ny, nx) in shapes]
    xrs = [x.reshape(x.shape[0], x.shape[1], -1) for x in (x0, x1, x2)]
    wts = [W.T for W in (W0, W1, W2)]
    br = jnp.stack([b0, b1, b2]).reshape(3, 1, NA * NO)

    outs = pl.pallas_call(
        _detect_kernel,
        grid=(bs,),
        in_specs=[
            pl.BlockSpec((1, shapes[0][1], hws[0]), lambda g: (g, 0, 0)),
            pl.BlockSpec((1, shapes[1][1], hws[1]), lambda g: (g, 0, 0)),
            pl.BlockSpec((1, shapes[2][1], hws[2]), lambda g: (g, 0, 0)),
            pl.BlockSpec((shapes[0][1], NA * NO), lambda g: (0, 0)),
            pl.BlockSpec((shapes[1][1], NA * NO), lambda g: (0, 0)),
            pl.BlockSpec((shapes[2][1], NA * NO), lambda g: (0, 0)),
            pl.BlockSpec((3, 1, NA * NO), lambda g: (0, 0, 0)),
        ],
        out_specs=[
            pl.BlockSpec((1, NA, hws[0], NO), lambda g: (g, 0, 0, 0)),
            pl.BlockSpec((1, NA, hws[1], NO), lambda g: (g, 0, 0, 0)),
            pl.BlockSpec((1, NA, hws[2], NO), lambda g: (g, 0, 0, 0)),
        ],
        out_shape=[
            jax.ShapeDtypeStruct((bs, NA, hws[0], NO), jnp.float32),
            jax.ShapeDtypeStruct((bs, NA, hws[1], NO), jnp.float32),
            jax.ShapeDtypeStruct((bs, NA, hws[2], NO), jnp.float32),
        ],
        compiler_params=pltpu.CompilerParams(
            dimension_semantics=("parallel",)),
    )(xrs[0], xrs[1], xrs[2], wts[0], wts[1], wts[2], br)
    return tuple(
        o.reshape(bs, NA, ny, nx, NO)
        for o, (_, _, ny, nx) in zip(outs, shapes))
